# baseline (speedup 1.0000x reference)
import functools

import jax
import jax.numpy as jnp
from jax import lax
from jax.experimental import pallas as pl
from jax.experimental.pallas import tpu as pltpu

F32 = jnp.float32

C = 512
NROW = 1000


def _round_up(x, m):
    return (x + m - 1) // m * m


def _wrel_body(c_ref, w_ref, o_ref):
    o_ref[0] = jnp.dot(c_ref[0], w_ref[0], preferred_element_type=F32)


def _wrel(c_all, w_all):
    L, R, _ = c_all.shape
    K = w_all.shape[-1]
    return pl.pallas_call(
        _wrel_body,
        grid=(L,),
        in_specs=[
            pl.BlockSpec((1, R, R), lambda l: (l, 0, 0)),
            pl.BlockSpec((1, R, K), lambda l: (l, 0, 0)),
        ],
        out_specs=pl.BlockSpec((1, R, K), lambda l: (l, 0, 0)),
        out_shape=jax.ShapeDtypeStruct((L, R, K), F32),
    )(c_all, w_all)


def _msg_body(rel_ref, g_ref, w_ref, o_ref):
    o_ref[...] = jnp.dot(g_ref[...], w_ref[0], preferred_element_type=F32)


def _messages(chunk_rel, gathered, wrel):
    P, D = gathered.shape
    H = wrel.shape[-1]
    nchunks = P // C
    return pl.pallas_call(
        _msg_body,
        grid_spec=pltpu.PrefetchScalarGridSpec(
            num_scalar_prefetch=1,
            grid=(nchunks,),
            in_specs=[
                pl.BlockSpec((C, D), lambda i, rel: (i, 0)),
                pl.BlockSpec((1, D, H), lambda i, rel: (rel[i], 0, 0)),
            ],
            out_specs=pl.BlockSpec((C, H), lambda i, rel: (i, 0)),
        ),
        out_shape=jax.ShapeDtypeStruct((P, H), F32),
    )(chunk_rel, gathered, wrel)


def _comb_body(p_ref, b_ref, o_ref):
    o_ref[...] = jnp.maximum(p_ref[...] + b_ref[...], 0.0)


def _combine(agg, bias):
    N, H = agg.shape
    return pl.pallas_call(
        _comb_body,
        grid=(N // NROW,),
        in_specs=[
            pl.BlockSpec((NROW, H), lambda i: (i, 0)),
            pl.BlockSpec((1, H), lambda i: (0, 0)),
        ],
        out_specs=pl.BlockSpec((NROW, H), lambda i: (i, 0)),
        out_shape=jax.ShapeDtypeStruct((N, H), F32),
    )(agg, bias.reshape(1, H))


def _red_body(p_ref, b_ref, o_ref):
    i = pl.program_id(0)
    s = jnp.sum(jnp.maximum(p_ref[...] + b_ref[...], 0.0), axis=0, keepdims=True)

    @pl.when(i == 0)
    def _():
        o_ref[...] = s

    @pl.when(i > 0)
    def _():
        o_ref[...] += s


def _combine_reduce(agg, bias):
    N, H = agg.shape
    return pl.pallas_call(
        _red_body,
        grid=(N // NROW,),
        in_specs=[
            pl.BlockSpec((NROW, H), lambda i: (i, 0)),
            pl.BlockSpec((1, H), lambda i: (0, 0)),
        ],
        out_specs=pl.BlockSpec((1, H), lambda i: (0, 0)),
        out_shape=jax.ShapeDtypeStruct((1, H), F32),
    )(agg, bias.reshape(1, H))


def _fc_body(g_ref, w0_ref, b0_ref, w1_ref, b1_ref, w2_ref, b2_ref,
             pw_ref, pb_ref, o_ref):
    x = g_ref[...]
    x = jnp.maximum(jnp.dot(x, w0_ref[...], preferred_element_type=F32) + b0_ref[...], 0.0)
    x = jnp.maximum(jnp.dot(x, w1_ref[...], preferred_element_type=F32) + b1_ref[...], 0.0)
    x = jnp.maximum(jnp.dot(x, w2_ref[...], preferred_element_type=F32) + b2_ref[...], 0.0)
    o_ref[...] = jnp.dot(x, pw_ref[...], preferred_element_type=F32) + pb_ref[...]


def _fc_head(g, f0W, f0b, f1W, f1b, f2W, f2b, pW, pb):
    H = g.shape[-1]
    pW_pad = jnp.zeros((H, H), F32).at[:, : pW.shape[1]].set(pW)
    pb_pad = jnp.zeros((1, H), F32).at[0, : pb.shape[0]].set(pb)
    out = pl.pallas_call(
        _fc_body,
        out_shape=jax.ShapeDtypeStruct((1, H), F32),
    )(g, f0W, f0b.reshape(1, H), f1W, f1b.reshape(1, H),
      f2W, f2b.reshape(1, H), pW_pad, pb_pad)
    return out[:, : pW.shape[1]]


def _edge_slots(src, dst, et, num_rels, num_nodes):
    E = src.shape[0]
    P = _round_up(E + num_rels * C, 32 * 128)
    nchunks = P // C
    perm = jnp.argsort(et)
    et_s = jnp.take(et, perm)
    src_s = jnp.take(src, perm)
    dst_s = jnp.take(dst, perm)
    bounds = jnp.searchsorted(et_s, jnp.arange(num_rels + 1, dtype=jnp.int32))
    counts = (bounds[1:] - bounds[:-1]).astype(jnp.int32)
    off = bounds[:-1].astype(jnp.int32)
    pcounts = (counts + C - 1) // C * C
    poff = (jnp.cumsum(pcounts) - pcounts).astype(jnp.int32)
    slot = jnp.take(poff, et_s) + jnp.arange(E, dtype=jnp.int32) - jnp.take(off, et_s)
    slot_src = jnp.zeros((P,), jnp.int32).at[slot].set(src_s)
    slot_dst = jnp.full((P,), num_nodes, jnp.int32).at[slot].set(dst_s)
    chunk_rel = jnp.clip(
        jnp.searchsorted(poff, jnp.arange(nchunks, dtype=jnp.int32) * C, side="right") - 1,
        0, num_rels - 1,
    ).astype(jnp.int32)
    return slot_src, slot_dst, chunk_rel, P


def kernel(node_feats, edge_index, edge_feats,
           w0, c0, b0, w1, c1, b1, w2, c2, b2,
           f0W, f0b, f1W, f1b, f2W, f2b, pW, pb):
    N, D = node_feats.shape
    R = w0.shape[0]
    H = w0.shape[2]
    src = edge_index[0]
    dst = edge_index[1]
    et = edge_feats.astype(jnp.int32)

    slot_src, slot_dst, chunk_rel, P = _edge_slots(src, dst, et, R, N)

    w_all = jnp.stack([w0.reshape(R, -1), w1.reshape(R, -1), w2.reshape(R, -1)])
    c_all = jnp.stack([c0, c1, c2])
    wrel_all = _wrel(c_all, w_all)

    h = node_feats
    g = None
    for l, bias in enumerate((b0, b1, b2)):
        wrel = wrel_all[l].reshape(R, D, H)
        gathered = jnp.take(h, slot_src, axis=0)
        msg = _messages(chunk_rel, gathered, wrel)
        agg = jax.ops.segment_sum(msg, slot_dst, num_segments=N + 16)[:N]
        if l < 2:
            h = _combine(agg, bias)
        else:
            g = _combine_reduce(agg, bias)

    return _fc_head(g, f0W, f0b, f1W, f1b, f2W, f2b, pW, pb)


# speedup vs baseline: 1.9868x; 1.9868x over previous
import functools

import jax
import jax.numpy as jnp
from jax import lax
from jax.experimental import pallas as pl
from jax.experimental.pallas import tpu as pltpu
from jax.experimental.pallas import tpu_sc as plsc

F32 = jnp.float32
I32 = jnp.int32

C = 512
BATCH = 128
NC = 2
NS = 16
NW = NC * NS


def _round_up(x, m):
    return (x + m - 1) // m * m


def _wrel_body(c_ref, w_ref, o_ref):
    o_ref[0] = jnp.dot(c_ref[0], w_ref[0], preferred_element_type=F32)


def _wrel(c_all, w_all):
    L, R, _ = c_all.shape
    K = w_all.shape[-1]
    return pl.pallas_call(
        _wrel_body,
        grid=(L,),
        in_specs=[
            pl.BlockSpec((1, R, R), lambda l: (l, 0, 0)),
            pl.BlockSpec((1, R, K), lambda l: (l, 0, 0)),
        ],
        out_specs=pl.BlockSpec((1, R, K), lambda l: (l, 0, 0)),
        out_shape=jax.ShapeDtypeStruct((L, R, K), F32),
    )(c_all, w_all)


def _sc_gather(h, src_ext, e_slot3):
    _, nb, _ = e_slot3.shape
    P = NW * nb * BATCH
    D = h.shape[1]
    per_w = P // NW
    mesh = plsc.VectorSubcoreMesh(core_axis_name="c", subcore_axis_name="s")

    @functools.partial(
        pl.kernel,
        mesh=mesh,
        out_type=jax.ShapeDtypeStruct((P, D), F32),
        scratch_types=[
            pltpu.VMEM((nb, BATCH), I32),
            pltpu.VMEM((BATCH,), I32),
            pltpu.VMEM((BATCH, D), F32),
            pltpu.SemaphoreType.DMA,
        ],
    )
    def k(h_hbm, src_hbm, eslot_hbm, out_hbm, e_v, sidx_v, rows_v, sem):
        wid = lax.axis_index("s") * NC + lax.axis_index("c")
        base = wid * per_w
        pltpu.sync_copy(eslot_hbm.at[wid], e_v)

        def body(b, _):
            pltpu.async_copy(src_hbm.at[e_v.at[b]], sidx_v, sem).wait()
            pltpu.async_copy(h_hbm.at[sidx_v], rows_v, sem).wait()
            pltpu.sync_copy(rows_v, out_hbm.at[pl.ds(base + b * BATCH, BATCH)])
            return 0

        lax.fori_loop(0, nb, body, 0)

    return k(h, src_ext, e_slot3)


def _sc_scatter(msg, dst_ext, e_slot3, zeros_acc, acc_rows):
    P, H = msg.shape
    per_w = P // NW
    nb = per_w // BATCH
    stripe = acc_rows // NS
    mesh = plsc.VectorSubcoreMesh(core_axis_name="c", subcore_axis_name="s")

    @functools.partial(
        pl.kernel,
        mesh=mesh,
        out_type=jax.ShapeDtypeStruct((NC, acc_rows, H), F32),
        scratch_types=[
            pltpu.VMEM_SHARED((acc_rows, H), F32),
            pltpu.VMEM((nb, BATCH), I32),
            pltpu.VMEM((BATCH,), I32),
            pltpu.VMEM((BATCH, H), F32),
            pltpu.SemaphoreType.DMA,
        ],
    )
    def k(msg_hbm, dst_hbm, eslot_hbm, zeros_hbm, out_hbm,
          acc, e_v, didx_v, rows_v, sem):
        cid = lax.axis_index("c")
        sid = lax.axis_index("s")
        wid = sid * NC + cid
        base = wid * per_w
        r0 = sid * stripe
        pltpu.sync_copy(zeros_hbm.at[pl.ds(r0, stripe)],
                        acc.at[pl.ds(r0, stripe)])
        pltpu.sync_copy(eslot_hbm.at[wid], e_v)
        plsc.subcore_barrier()

        def body(b, _):
            pltpu.async_copy(dst_hbm.at[e_v.at[b]], didx_v, sem).wait()
            pltpu.sync_copy(msg_hbm.at[pl.ds(base + b * BATCH, BATCH)], rows_v)
            pltpu.sync_copy(rows_v, acc.at[didx_v], add=True)
            return 0

        lax.fori_loop(0, nb, body, 0)
        plsc.subcore_barrier()
        pltpu.sync_copy(acc.at[pl.ds(r0, stripe)],
                        out_hbm.at[cid, pl.ds(r0, stripe)])

    return k(msg, dst_ext, e_slot3, zeros_acc)


def _msg_body(rel_ref, g_ref, w_ref, o_ref):
    o_ref[...] = jnp.dot(g_ref[...], w_ref[0], preferred_element_type=F32)


def _messages(chunk_rel, gathered, wrel):
    P, D = gathered.shape
    H = wrel.shape[-1]
    nchunks = P // C
    return pl.pallas_call(
        _msg_body,
        grid_spec=pltpu.PrefetchScalarGridSpec(
            num_scalar_prefetch=1,
            grid=(nchunks,),
            in_specs=[
                pl.BlockSpec((C, D), lambda i, rel: (i, 0)),
                pl.BlockSpec((1, D, H), lambda i, rel: (rel[i], 0, 0)),
            ],
            out_specs=pl.BlockSpec((C, H), lambda i, rel: (i, 0)),
        ),
        out_shape=jax.ShapeDtypeStruct((P, H), F32),
    )(chunk_rel, gathered, wrel)


def _comb_body(p_ref, b_ref, o_ref):
    o_ref[...] = jnp.maximum(p_ref[0] + p_ref[1] + b_ref[...], 0.0)


def _combine(partials, bias, nrow):
    _, A, H = partials.shape
    return pl.pallas_call(
        _comb_body,
        grid=(A // nrow,),
        in_specs=[
            pl.BlockSpec((2, nrow, H), lambda i: (0, i, 0)),
            pl.BlockSpec((1, H), lambda i: (0, 0)),
        ],
        out_specs=pl.BlockSpec((nrow, H), lambda i: (i, 0)),
        out_shape=jax.ShapeDtypeStruct((A, H), F32),
    )(partials, bias.reshape(1, H))


def _red_body(n_nodes, nrow, p_ref, b_ref, o_ref):
    i = pl.program_id(0)
    row = i * nrow + lax.broadcasted_iota(I32, p_ref.shape[1:], 0)
    h = jnp.maximum(p_ref[0] + p_ref[1] + b_ref[...], 0.0)
    h = jnp.where(row < n_nodes, h, 0.0)
    s = jnp.sum(h, axis=0, keepdims=True)

    @pl.when(i == 0)
    def _():
        o_ref[...] = s

    @pl.when(i > 0)
    def _():
        o_ref[...] += s


def _combine_reduce(partials, bias, n_nodes, nrow):
    _, A, H = partials.shape
    return pl.pallas_call(
        functools.partial(_red_body, n_nodes, nrow),
        grid=(A // nrow,),
        in_specs=[
            pl.BlockSpec((2, nrow, H), lambda i: (0, i, 0)),
            pl.BlockSpec((1, H), lambda i: (0, 0)),
        ],
        out_specs=pl.BlockSpec((1, H), lambda i: (0, 0)),
        out_shape=jax.ShapeDtypeStruct((1, H), F32),
    )(partials, bias.reshape(1, H))


def _fc_body(g_ref, w0_ref, b0_ref, w1_ref, b1_ref, w2_ref, b2_ref,
             pw_ref, pb_ref, o_ref):
    x = g_ref[...]
    x = jnp.maximum(jnp.dot(x, w0_ref[...], preferred_element_type=F32) + b0_ref[...], 0.0)
    x = jnp.maximum(jnp.dot(x, w1_ref[...], preferred_element_type=F32) + b1_ref[...], 0.0)
    x = jnp.maximum(jnp.dot(x, w2_ref[...], preferred_element_type=F32) + b2_ref[...], 0.0)
    o_ref[...] = jnp.dot(x, pw_ref[...], preferred_element_type=F32) + pb_ref[...]


def _fc_head(g, f0W, f0b, f1W, f1b, f2W, f2b, pW, pb):
    H = g.shape[-1]
    pW_pad = jnp.zeros((H, H), F32).at[:, : pW.shape[1]].set(pW)
    pb_pad = jnp.zeros((1, H), F32).at[0, : pb.shape[0]].set(pb)
    out = pl.pallas_call(
        _fc_body,
        out_shape=jax.ShapeDtypeStruct((1, H), F32),
    )(g, f0W, f0b.reshape(1, H), f1W, f1b.reshape(1, H),
      f2W, f2b.reshape(1, H), pW_pad, pb_pad)
    return out[:, : pW.shape[1]]


def _edge_slots(src, dst, et, num_rels, sentinel_dst):
    E = src.shape[0]
    P = _round_up(E + num_rels * C, NW * BATCH)
    nchunks = P // C
    et_s, src_s, dst_s = lax.sort((et, src, dst), num_keys=1)
    src_ext = jnp.concatenate([src_s, jnp.zeros((8,), I32)])
    dst_ext = jnp.concatenate([dst_s, jnp.full((8,), sentinel_dst, I32)])
    bounds = jnp.searchsorted(et_s, jnp.arange(num_rels + 1, dtype=I32)).astype(I32)
    counts = bounds[1:] - bounds[:-1]
    off = bounds[:-1]
    pcounts = (counts + C - 1) // C * C
    poff = (jnp.cumsum(pcounts) - pcounts).astype(I32)
    chunk_rel = jnp.clip(
        jnp.searchsorted(poff, jnp.arange(nchunks, dtype=I32) * C, side="right") - 1,
        0, num_rels - 1,
    ).astype(I32)
    shift = jnp.repeat(jnp.take(off, chunk_rel) - jnp.take(poff, chunk_rel), C)
    cnt_rep = jnp.repeat(jnp.take(counts, chunk_rel), C)
    pos = jnp.arange(P, dtype=I32)
    e_sorted = pos + shift
    valid = (e_sorted - jnp.repeat(jnp.take(off, chunk_rel), C)) < cnt_rep
    e_slot = jnp.where(valid, e_sorted, E)
    nb = P // (NW * BATCH)
    return src_ext, dst_ext, e_slot.reshape(NW, nb, BATCH), chunk_rel, P


def kernel(node_feats, edge_index, edge_feats,
           w0, c0, b0, w1, c1, b1, w2, c2, b2,
           f0W, f0b, f1W, f1b, f2W, f2b, pW, pb):
    N, D = node_feats.shape
    R = w0.shape[0]
    H = w0.shape[2]
    A = _round_up(N + 16, 1024)
    NROW = 1024
    src = edge_index[0].astype(I32)
    dst = edge_index[1].astype(I32)
    et = edge_feats.astype(I32)

    src_ext, dst_ext, e_slot, chunk_rel, P = _edge_slots(src, dst, et, R, N)
    zeros_acc = jnp.zeros((A, H), F32)

    w_all = jnp.stack([w0.reshape(R, -1), w1.reshape(R, -1), w2.reshape(R, -1)])
    c_all = jnp.stack([c0, c1, c2])
    wrel_all = _wrel(c_all, w_all)

    h = node_feats
    g = None
    for l, bias in enumerate((b0, b1, b2)):
        wrel = wrel_all[l].reshape(R, D, H)
        gathered = _sc_gather(h, src_ext, e_slot)
        msg = _messages(chunk_rel, gathered, wrel)
        partials = _sc_scatter(msg, dst_ext, e_slot, zeros_acc, A)
        if l < 2:
            h = _combine(partials, bias, NROW)
        else:
            g = _combine_reduce(partials, bias, N, NROW)

    return _fc_head(g, f0W, f0b, f1W, f1b, f2W, f2b, pW, pb)


# speedup vs baseline: 4.3129x; 2.1708x over previous
import functools

import jax
import jax.numpy as jnp
from jax import lax
from jax.experimental import pallas as pl
from jax.experimental.pallas import tpu as pltpu
from jax.experimental.pallas import tpu_sc as plsc

F32 = jnp.float32
I32 = jnp.int32

C = 512
BATCH = 128
NC = 2
NS = 16
NW = NC * NS


def _round_up(x, m):
    return (x + m - 1) // m * m


def _wrel_body(c_ref, w_ref, o_ref):
    o_ref[0] = jnp.dot(c_ref[0], w_ref[0], preferred_element_type=F32)


def _wrel(c_all, w_all):
    L, R, _ = c_all.shape
    K = w_all.shape[-1]
    return pl.pallas_call(
        _wrel_body,
        grid=(L,),
        in_specs=[
            pl.BlockSpec((1, R, R), lambda l: (l, 0, 0)),
            pl.BlockSpec((1, R, K), lambda l: (l, 0, 0)),
        ],
        out_specs=pl.BlockSpec((1, R, K), lambda l: (l, 0, 0)),
        out_shape=jax.ShapeDtypeStruct((L, R, K), F32),
    )(c_all, w_all)


def _sc_gather(h, src_ext, e_slot3):
    _, nb, _ = e_slot3.shape
    P = NW * nb * BATCH
    Nh, D = h.shape
    EX = src_ext.shape[0]
    per_w = P // NW
    h_stripe = Nh // NS
    s_stripe = EX // NS
    mesh = plsc.VectorSubcoreMesh(core_axis_name="c", subcore_axis_name="s")

    @functools.partial(
        pl.kernel,
        mesh=mesh,
        out_type=jax.ShapeDtypeStruct((P, D), F32),
        scratch_types=[
            pltpu.VMEM_SHARED((Nh, D), F32),
            pltpu.VMEM((nb, BATCH), I32),
            pltpu.VMEM((BATCH,), I32),
            pltpu.VMEM((BATCH, D), F32),
            pltpu.SemaphoreType.DMA,
        ],
    )
    def k(h_hbm, src_hbm, eslot_hbm, out_hbm, h_sh, e_v, sidx_v,
          rows_v, sem):
        sid = lax.axis_index("s")
        wid = sid * NC + lax.axis_index("c")
        base = wid * per_w
        pltpu.sync_copy(h_hbm.at[pl.ds(sid * h_stripe, h_stripe)],
                        h_sh.at[pl.ds(sid * h_stripe, h_stripe)])
        pltpu.sync_copy(eslot_hbm.at[wid], e_v)
        plsc.subcore_barrier()

        def body(b, _):
            pltpu.async_copy(src_hbm.at[e_v.at[b]], sidx_v, sem).wait()
            pltpu.async_copy(h_sh.at[sidx_v], rows_v, sem).wait()
            pltpu.sync_copy(rows_v, out_hbm.at[pl.ds(base + b * BATCH, BATCH)])
            return 0

        lax.fori_loop(0, nb, body, 0)

    return k(h, src_ext, e_slot3)


def _sc_scatter(msg, dst_ext, e_slot3, zeros_acc, acc_rows):
    P, H = msg.shape
    per_w = P // NW
    nb = per_w // BATCH
    stripe = acc_rows // NS
    mesh = plsc.VectorSubcoreMesh(core_axis_name="c", subcore_axis_name="s")

    @functools.partial(
        pl.kernel,
        mesh=mesh,
        out_type=jax.ShapeDtypeStruct((NC, acc_rows, H), F32),
        scratch_types=[
            pltpu.VMEM_SHARED((acc_rows, H), F32),
            pltpu.VMEM((nb, BATCH), I32),
            pltpu.VMEM((BATCH,), I32),
            pltpu.VMEM((BATCH, H), F32),
            pltpu.SemaphoreType.DMA,
        ],
    )
    def k(msg_hbm, dst_hbm, eslot_hbm, zeros_hbm, out_hbm,
          acc, e_v, didx_v, rows_v, sem):
        cid = lax.axis_index("c")
        sid = lax.axis_index("s")
        wid = sid * NC + cid
        base = wid * per_w
        r0 = sid * stripe
        pltpu.sync_copy(zeros_hbm.at[pl.ds(r0, stripe)],
                        acc.at[pl.ds(r0, stripe)])
        pltpu.sync_copy(eslot_hbm.at[wid], e_v)
        plsc.subcore_barrier()

        def body(b, _):
            pltpu.async_copy(dst_hbm.at[e_v.at[b]], didx_v, sem).wait()
            pltpu.sync_copy(msg_hbm.at[pl.ds(base + b * BATCH, BATCH)], rows_v)
            pltpu.sync_copy(rows_v, acc.at[didx_v], add=True)
            return 0

        lax.fori_loop(0, nb, body, 0)
        plsc.subcore_barrier()
        pltpu.sync_copy(acc.at[pl.ds(r0, stripe)],
                        out_hbm.at[cid, pl.ds(r0, stripe)])

    return k(msg, dst_ext, e_slot3, zeros_acc)


def _msg_body(rel_ref, g_ref, w_ref, o_ref):
    o_ref[...] = jnp.dot(g_ref[...], w_ref[0], preferred_element_type=F32)


def _messages(chunk_rel, gathered, wrel):
    P, D = gathered.shape
    H = wrel.shape[-1]
    nchunks = P // C
    return pl.pallas_call(
        _msg_body,
        grid_spec=pltpu.PrefetchScalarGridSpec(
            num_scalar_prefetch=1,
            grid=(nchunks,),
            in_specs=[
                pl.BlockSpec((C, D), lambda i, rel: (i, 0)),
                pl.BlockSpec((1, D, H), lambda i, rel: (rel[i], 0, 0)),
            ],
            out_specs=pl.BlockSpec((C, H), lambda i, rel: (i, 0)),
        ),
        out_shape=jax.ShapeDtypeStruct((P, H), F32),
    )(chunk_rel, gathered, wrel)


def _comb_body(p_ref, b_ref, o_ref):
    o_ref[...] = jnp.maximum(p_ref[0] + p_ref[1] + b_ref[...], 0.0)


def _combine(partials, bias, nrow):
    _, A, H = partials.shape
    return pl.pallas_call(
        _comb_body,
        grid=(A // nrow,),
        in_specs=[
            pl.BlockSpec((2, nrow, H), lambda i: (0, i, 0)),
            pl.BlockSpec((1, H), lambda i: (0, 0)),
        ],
        out_specs=pl.BlockSpec((nrow, H), lambda i: (i, 0)),
        out_shape=jax.ShapeDtypeStruct((A, H), F32),
    )(partials, bias.reshape(1, H))


def _red_body(n_nodes, nrow, p_ref, b_ref, o_ref):
    i = pl.program_id(0)
    row = i * nrow + lax.broadcasted_iota(I32, p_ref.shape[1:], 0)
    h = jnp.maximum(p_ref[0] + p_ref[1] + b_ref[...], 0.0)
    h = jnp.where(row < n_nodes, h, 0.0)
    s = jnp.sum(h, axis=0, keepdims=True)

    @pl.when(i == 0)
    def _():
        o_ref[...] = s

    @pl.when(i > 0)
    def _():
        o_ref[...] += s


def _combine_reduce(partials, bias, n_nodes, nrow):
    _, A, H = partials.shape
    return pl.pallas_call(
        functools.partial(_red_body, n_nodes, nrow),
        grid=(A // nrow,),
        in_specs=[
            pl.BlockSpec((2, nrow, H), lambda i: (0, i, 0)),
            pl.BlockSpec((1, H), lambda i: (0, 0)),
        ],
        out_specs=pl.BlockSpec((1, H), lambda i: (0, 0)),
        out_shape=jax.ShapeDtypeStruct((1, H), F32),
    )(partials, bias.reshape(1, H))


def _fc_body(g_ref, w0_ref, b0_ref, w1_ref, b1_ref, w2_ref, b2_ref,
             pw_ref, pb_ref, o_ref):
    x = g_ref[...]
    x = jnp.maximum(jnp.dot(x, w0_ref[...], preferred_element_type=F32) + b0_ref[...], 0.0)
    x = jnp.maximum(jnp.dot(x, w1_ref[...], preferred_element_type=F32) + b1_ref[...], 0.0)
    x = jnp.maximum(jnp.dot(x, w2_ref[...], preferred_element_type=F32) + b2_ref[...], 0.0)
    o_ref[...] = jnp.dot(x, pw_ref[...], preferred_element_type=F32) + pb_ref[...]


def _fc_head(g, f0W, f0b, f1W, f1b, f2W, f2b, pW, pb):
    H = g.shape[-1]
    pW_pad = jnp.zeros((H, H), F32).at[:, : pW.shape[1]].set(pW)
    pb_pad = jnp.zeros((1, H), F32).at[0, : pb.shape[0]].set(pb)
    out = pl.pallas_call(
        _fc_body,
        out_shape=jax.ShapeDtypeStruct((1, H), F32),
    )(g, f0W, f0b.reshape(1, H), f1W, f1b.reshape(1, H),
      f2W, f2b.reshape(1, H), pW_pad, pb_pad)
    return out[:, : pW.shape[1]]


def _edge_slots(src, dst, et, num_rels, sentinel_dst):
    E = src.shape[0]
    P = _round_up(E + num_rels * C, NW * BATCH)
    nchunks = P // C
    et_s, src_s, dst_s = lax.sort((et, src, dst), num_keys=1)
    npad = _round_up(E + 1, 256) - E
    src_ext = jnp.concatenate([src_s, jnp.zeros((npad,), I32)])
    dst_ext = jnp.concatenate([dst_s, jnp.full((npad,), sentinel_dst, I32)])
    bounds = jnp.searchsorted(et_s, jnp.arange(num_rels + 1, dtype=I32)).astype(I32)
    counts = bounds[1:] - bounds[:-1]
    off = bounds[:-1]
    pcounts = (counts + C - 1) // C * C
    poff = (jnp.cumsum(pcounts) - pcounts).astype(I32)
    chunk_rel = jnp.clip(
        jnp.searchsorted(poff, jnp.arange(nchunks, dtype=I32) * C, side="right") - 1,
        0, num_rels - 1,
    ).astype(I32)
    shift = jnp.repeat(jnp.take(off, chunk_rel) - jnp.take(poff, chunk_rel), C)
    cnt_rep = jnp.repeat(jnp.take(counts, chunk_rel), C)
    pos = jnp.arange(P, dtype=I32)
    e_sorted = pos + shift
    valid = (e_sorted - jnp.repeat(jnp.take(off, chunk_rel), C)) < cnt_rep
    e_slot = jnp.where(valid, e_sorted, E)
    nb = P // (NW * BATCH)
    return src_ext, dst_ext, e_slot.reshape(NW, nb, BATCH), chunk_rel, P


def kernel(node_feats, edge_index, edge_feats,
           w0, c0, b0, w1, c1, b1, w2, c2, b2,
           f0W, f0b, f1W, f1b, f2W, f2b, pW, pb):
    N, D = node_feats.shape
    R = w0.shape[0]
    H = w0.shape[2]
    A = _round_up(N + 16, 1024)
    NROW = 1024
    src = edge_index[0].astype(I32)
    dst = edge_index[1].astype(I32)
    et = edge_feats.astype(I32)

    src_ext, dst_ext, e_slot, chunk_rel, P = _edge_slots(src, dst, et, R, N)
    zeros_acc = jnp.zeros((A, H), F32)

    w_all = jnp.stack([w0.reshape(R, -1), w1.reshape(R, -1), w2.reshape(R, -1)])
    c_all = jnp.stack([c0, c1, c2])
    wrel_all = _wrel(c_all, w_all)

    h = jnp.pad(node_feats, ((0, A - N), (0, 0)))
    g = None
    for l, bias in enumerate((b0, b1, b2)):
        wrel = wrel_all[l].reshape(R, D, H)
        gathered = _sc_gather(h, src_ext, e_slot)
        msg = _messages(chunk_rel, gathered, wrel)
        partials = _sc_scatter(msg, dst_ext, e_slot, zeros_acc, A)
        if l < 2:
            h = _combine(partials, bias, NROW)
        else:
            g = _combine_reduce(partials, bias, N, NROW)

    return _fc_head(g, f0W, f0b, f1W, f1b, f2W, f2b, pW, pb)
